# baseline (speedup 1.0000x reference)
import functools

import jax
import jax.numpy as jnp
from jax.experimental import pallas as pl
from jax.experimental.pallas import tpu as pltpu


def _copy_dma(w_hbm, o_hbm, vmem, in_sems, out_sems, *, n_chunks, rows):
    for c in range(n_chunks):
        pltpu.make_async_copy(
            w_hbm.at[pl.ds(c * rows, rows), :],
            vmem.at[pl.ds(c * rows, rows), :],
            in_sems.at[c],
        ).start()
    for c in range(n_chunks):
        pltpu.make_async_copy(
            w_hbm.at[pl.ds(c * rows, rows), :],
            vmem.at[pl.ds(c * rows, rows), :],
            in_sems.at[c],
        ).wait()
        pltpu.make_async_copy(
            vmem.at[pl.ds(c * rows, rows), :],
            o_hbm.at[pl.ds(c * rows, rows), :],
            out_sems.at[c],
        ).start()
    for c in range(n_chunks):
        pltpu.make_async_copy(
            vmem.at[pl.ds(c * rows, rows), :],
            o_hbm.at[pl.ds(c * rows, rows), :],
            out_sems.at[c],
        ).wait()


def kernel(x, W):
    del x
    S, D = W.shape
    n_chunks = 16
    rows = S // n_chunks
    return pl.pallas_call(
        functools.partial(_copy_dma, n_chunks=n_chunks, rows=rows),
        in_specs=[pl.BlockSpec(memory_space=pl.ANY)],
        out_specs=pl.BlockSpec(memory_space=pl.ANY),
        out_shape=jax.ShapeDtypeStruct((S, D), W.dtype),
        scratch_shapes=[
            pltpu.VMEM((S, D), W.dtype),
            pltpu.SemaphoreType.DMA((n_chunks,)),
            pltpu.SemaphoreType.DMA((n_chunks,)),
        ],
    )(W)


# speedup vs baseline: 1.0338x; 1.0338x over previous
import functools

import jax
import jax.numpy as jnp
from jax.experimental import pallas as pl
from jax.experimental.pallas import tpu as pltpu


def _copy_dma(w_hbm, o_hbm, vmem, in_sems, out_sems, *, n_chunks, rows):
    for c in range(n_chunks):
        pltpu.make_async_copy(
            w_hbm.at[pl.ds(c * rows, rows), :],
            vmem.at[pl.ds(c * rows, rows), :],
            in_sems.at[c],
        ).start()
    for c in range(n_chunks):
        pltpu.make_async_copy(
            w_hbm.at[pl.ds(c * rows, rows), :],
            vmem.at[pl.ds(c * rows, rows), :],
            in_sems.at[c],
        ).wait()
        pltpu.make_async_copy(
            vmem.at[pl.ds(c * rows, rows), :],
            o_hbm.at[pl.ds(c * rows, rows), :],
            out_sems.at[c],
        ).start()
    for c in range(n_chunks):
        pltpu.make_async_copy(
            vmem.at[pl.ds(c * rows, rows), :],
            o_hbm.at[pl.ds(c * rows, rows), :],
            out_sems.at[c],
        ).wait()


def kernel(x, W):
    del x
    S, D = W.shape
    n_chunks = 2
    rows = S // n_chunks
    return pl.pallas_call(
        functools.partial(_copy_dma, n_chunks=n_chunks, rows=rows),
        in_specs=[pl.BlockSpec(memory_space=pl.ANY)],
        out_specs=pl.BlockSpec(memory_space=pl.ANY),
        out_shape=jax.ShapeDtypeStruct((S, D), W.dtype),
        scratch_shapes=[
            pltpu.VMEM((S, D), W.dtype),
            pltpu.SemaphoreType.DMA((n_chunks,)),
            pltpu.SemaphoreType.DMA((n_chunks,)),
        ],
    )(W)
